# baseline (speedup 1.0000x reference)
import functools

import jax
import jax.numpy as jnp
from jax import lax
from jax.experimental import pallas as pl
from jax.experimental.pallas import tpu as pltpu
from jax.experimental.pallas import tpu_sc as plsc

N = 10000
E = 320000
D = 128
H = 64
SLOPE = 0.2
NC, NS, L = 2, 16, 16
EPT = E // NS
B = 80
NBLK = EPT // B
NGB = E // B
U = 4
NJ = (NBLK // U) * U // U
NB_MAIN = NJ * U
RPT = N // NS
RB = 40
NRB = RPT // RB
RTAIL = RPT - NRB * RB


def _tables_body(x_ref, wl_ref, bl_ref, wr_ref, br_ref,
                 st0_ref, st1_ref, dt0_ref, dt1_ref):
  xb = x_ref[...]
  xl = lax.dot_general(xb, wl_ref[...], (((1,), (1,)), ((), ())),
                       preferred_element_type=jnp.float32) + bl_ref[...]
  xr = lax.dot_general(xb, wr_ref[...], (((1,), (1,)), ((), ())),
                       preferred_element_type=jnp.float32) + br_ref[...]
  st0_ref[:, 0:H] = xb[:, 0:H]
  st0_ref[:, H:D] = xl[:, 0:H]
  st1_ref[:, 0:H] = xb[:, H:D]
  st1_ref[:, H:D] = xl[:, H:D]
  dt0_ref[...] = xr[:, 0:H]
  dt1_ref[...] = xr[:, H:D]


def _make_tables(x, W_l, b_l, W_r, b_r):
  R = 1000
  return pl.pallas_call(
      _tables_body,
      grid=(N // R,),
      in_specs=[
          pl.BlockSpec((R, D), lambda i: (i, 0)),
          pl.BlockSpec((D, D), lambda i: (0, 0)),
          pl.BlockSpec((D,), lambda i: (0,)),
          pl.BlockSpec((D, D), lambda i: (0, 0)),
          pl.BlockSpec((D,), lambda i: (0,)),
      ],
      out_specs=[
          pl.BlockSpec((R, D), lambda i: (i, 0)),
          pl.BlockSpec((R, D), lambda i: (i, 0)),
          pl.BlockSpec((R, H), lambda i: (i, 0)),
          pl.BlockSpec((R, H), lambda i: (i, 0)),
      ],
      out_shape=[
          jax.ShapeDtypeStruct((N, D), jnp.float32),
          jax.ShapeDtypeStruct((N, D), jnp.float32),
          jax.ShapeDtypeStruct((N, H), jnp.float32),
          jax.ShapeDtypeStruct((N, H), jnp.float32),
      ],
  )(x, W_l, b_l, W_r, b_r)


def _sc_body(st0, st1, dt0, dt1, sdp, alpha, h0, h1,
             acc, ixp0, ixp1, ixp2, ixp3,
             g1a, g1b, g2a, g2b, aba, abb, oba, obb, hbuf,
             ix0, ix1, in0, in1, sc0, sc1):
  c = lax.axis_index("c")
  s = lax.axis_index("s")
  row0 = s * RPT
  eb = s * EPT
  ixp = (ixp0, ixp1, ixp2, ixp3)
  g1 = (g1a, g1b)
  g2 = (g2a, g2b)
  ab = (aba, abb)
  ob = (oba, obb)
  ixs = (ix0, ix1)
  ins = (in0, in1)
  scs = (sc0, sc1)

  def gblk(i):
    return jnp.minimum(s * NBLK + i, NGB - 1)

  @pl.loop(0, RB * (D // L))
  def _zfill(k):
    r = k // (D // L)
    cc = (k % (D // L)) * L
    oba[r, pl.ds(cc, L)] = jnp.zeros((L,), jnp.float32)

  @pl.loop(0, RPT // RB)
  def _zcopy(j):
    pltpu.sync_copy(oba.at[pl.ds(0, RB)], acc.at[pl.ds(row0 + j * RB, RB)])

  pltpu.sync_copy(oba.at[pl.ds(0, RTAIL)],
                  acc.at[pl.ds(row0 + NRB * RB, RTAIL)])

  plsc.subcore_barrier()

  for half in range(NC):
    @pl.when(c == half)
    def _half_program():
      st = st0 if half == 0 else st1
      dt = dt0 if half == 0 else dt1
      coff = half * H

      def compute_block(gin, din, ain, oout):
        @plsc.parallel_loop(0, B, unroll=4)
        def _row(r):
          for q in range(H // (2 * L)):
            c32 = q * 2 * L
            ilv = plsc.PackFormat.INTERLEAVED
            xa, xb = plsc.unpack(gin[r, pl.ds(c32, 2 * L)], format=ilv)
            xla, xlb = plsc.unpack(gin[r, pl.ds(H + c32, 2 * L)], format=ilv)
            xra, xrb = plsc.unpack(din[r, pl.ds(c32, 2 * L)], format=ilv)
            for t, (xv, xlv, xrv) in enumerate(((xa, xla, xra),
                                                (xb, xlb, xrb))):
              cc = c32 + t * L
              av = ain[r, pl.ds(cc, L)]
              z = (xlv + xrv) * av
              z = jnp.maximum(z, SLOPE * z)
              p = jnp.exp(z)
              oout[r, pl.ds(cc, L)] = xv * p
              oout[r, pl.ds(H + cc, L)] = p

      def emit_block(i, u):
        s2 = u % 2
        n2 = (u + 1) % 2
        pltpu.make_async_copy(sdp.at[0], ixp[(u + 1) % 4], ixs[n2]).wait()
        bn = gblk(i + 1) * B
        pltpu.async_copy(st.at[ixp[(u + 1) % 4].at[0]], g1[n2], ins[n2])
        pltpu.async_copy(dt.at[ixp[(u + 1) % 4].at[1]], g2[n2], ins[n2])
        pltpu.async_copy(alpha.at[pl.ds(bn, B), pl.ds(coff, H)],
                         ab[n2], ins[n2])
        pltpu.make_async_copy(st.at[pl.ds(0, B)], g1[s2], ins[s2]).wait()
        pltpu.make_async_copy(dt.at[pl.ds(0, B)], g2[s2], ins[s2]).wait()
        pltpu.make_async_copy(alpha.at[pl.ds(0, B), pl.ds(0, H)],
                              ab[s2], ins[s2]).wait()
        pltpu.make_async_copy(alpha.at[pl.ds(0, B), :], ob[s2],
                              scs[s2]).wait()
        compute_block(g1[s2], g2[s2], ab[s2], ob[s2])
        pltpu.async_copy(ob[s2], acc.at[ixp[u % 4].at[1]], scs[s2],
                         add=True)
        pltpu.async_copy(sdp.at[gblk(i + 2)], ixp[(u + 2) % 4], ixs[s2])

      pltpu.sync_copy(sdp.at[gblk(0)], ixp0)
      pltpu.async_copy(st.at[ixp0.at[0]], g1a, in0)
      pltpu.async_copy(dt.at[ixp0.at[1]], g2a, in0)
      pltpu.async_copy(alpha.at[pl.ds(eb, B), pl.ds(coff, H)], aba, in0)
      pltpu.async_copy(sdp.at[gblk(1)], ixp1, ix1)
      pltpu.async_copy(alpha.at[pl.ds(0, B), :], oba, sc0)
      pltpu.async_copy(alpha.at[pl.ds(0, B), :], obb, sc1)

      @pl.loop(0, NJ)
      def _super(j):
        i0 = j * U
        for u in range(U):
          emit_block(i0 + u, u)

      emit_block(NB_MAIN, 0)
      emit_block(NB_MAIN + 1, 1)

      pltpu.make_async_copy(alpha.at[pl.ds(0, B), :], ob[0], sc0).wait()
      pltpu.make_async_copy(alpha.at[pl.ds(0, B), :], ob[1], sc1).wait()
      pltpu.make_async_copy(st.at[pl.ds(0, B)], g1[0], in0).wait()
      pltpu.make_async_copy(dt.at[pl.ds(0, B)], g2[0], in0).wait()
      pltpu.make_async_copy(alpha.at[pl.ds(0, B), pl.ds(0, H)],
                            ab[0], in0).wait()
      pltpu.make_async_copy(sdp.at[0], ixp3, ix1).wait()

      plsc.subcore_barrier()

      hout = h0 if half == 0 else h1

      def div_chunk(rows, roff):
        pltpu.sync_copy(acc.at[pl.ds(roff, rows)], oba.at[pl.ds(0, rows)])

        @pl.loop(0, rows)
        def _erow(r):
          for q in range(H // L):
            cc = q * L
            num = oba[r, pl.ds(cc, L)]
            den = oba[r, pl.ds(H + cc, L)]
            hbuf[r, pl.ds(cc, L)] = jnp.where(den > 0.0, num / den, 0.0)

        pltpu.sync_copy(hbuf.at[pl.ds(0, rows)], hout.at[pl.ds(roff, rows)])

      @pl.loop(0, NRB)
      def _ep(k):
        div_chunk(RB, row0 + k * RB)

      div_chunk(RTAIL, row0 + NRB * RB)


_sc_edge = functools.partial(
    pl.kernel,
    out_type=[jax.ShapeDtypeStruct((N, H), jnp.float32),
              jax.ShapeDtypeStruct((N, H), jnp.float32)],
    mesh=plsc.VectorSubcoreMesh(core_axis_name="c", subcore_axis_name="s",
                                num_cores=NC, num_subcores=NS),
    compiler_params=pltpu.CompilerParams(use_tc_tiling_on_sc=False,
                                         needs_layout_passes=False),
    scratch_types=[
        pltpu.VMEM_SHARED((N, D), jnp.float32),
        pltpu.VMEM((2, B), jnp.int32),
        pltpu.VMEM((2, B), jnp.int32),
        pltpu.VMEM((2, B), jnp.int32),
        pltpu.VMEM((2, B), jnp.int32),
        pltpu.VMEM((B, D), jnp.bfloat16),
        pltpu.VMEM((B, D), jnp.bfloat16),
        pltpu.VMEM((B, H), jnp.bfloat16),
        pltpu.VMEM((B, H), jnp.bfloat16),
        pltpu.VMEM((B, H), jnp.float32),
        pltpu.VMEM((B, H), jnp.float32),
        pltpu.VMEM((B, D), jnp.float32),
        pltpu.VMEM((B, D), jnp.float32),
        pltpu.VMEM((RB, H), jnp.float32),
        pltpu.SemaphoreType.DMA,
        pltpu.SemaphoreType.DMA,
        pltpu.SemaphoreType.DMA,
        pltpu.SemaphoreType.DMA,
        pltpu.SemaphoreType.DMA,
        pltpu.SemaphoreType.DMA,
    ],
)(_sc_body)


def _ileave(t):
  n, w = t.shape
  t = t.reshape(n, w // 32, 2, 16).transpose(0, 1, 3, 2).reshape(n, w)
  return t.astype(jnp.bfloat16)


def kernel(x, edge_index, alpha, ntype, etype, W_l, b_l, W_r, b_r):
  ei = edge_index.astype(jnp.int32)
  sdp = jnp.stack([ei[0].reshape(NGB, B), ei[1].reshape(NGB, B)], axis=1)
  st0, st1, dt0, dt1 = _make_tables(x, W_l, b_l, W_r, b_r)
  h0, h1 = _sc_edge(_ileave(st0), _ileave(st1), _ileave(dt0), _ileave(dt1),
                    sdp, alpha)
  return jnp.concatenate([h0, h1], axis=1)


# speedup vs baseline: 1.0471x; 1.0471x over previous
import functools

import jax
import jax.numpy as jnp
from jax import lax
from jax.experimental import pallas as pl
from jax.experimental.pallas import tpu as pltpu
from jax.experimental.pallas import tpu_sc as plsc

N = 10000
E = 320000
D = 128
H = 64
SLOPE = 0.2
NC, NS, L = 2, 16, 16
EPT = E // NS
B = 80
NBLK = EPT // B
NGB = E // B
U = 4
NJ = (NBLK // U) * U // U
NB_MAIN = NJ * U
RPT = N // NS
RB = 40
NRB = RPT // RB
RTAIL = RPT - NRB * RB


def _tables_body(x_ref, wl_ref, bl_ref, wr_ref, br_ref,
                 st0_ref, st1_ref, dt0_ref, dt1_ref):
  xb = x_ref[...]
  xl = lax.dot_general(xb, wl_ref[...], (((1,), (1,)), ((), ())),
                       preferred_element_type=jnp.float32) + bl_ref[...]
  xr = lax.dot_general(xb, wr_ref[...], (((1,), (1,)), ((), ())),
                       preferred_element_type=jnp.float32) + br_ref[...]
  st0_ref[:, 0:H] = xb[:, 0:H]
  st0_ref[:, H:D] = xl[:, 0:H]
  st1_ref[:, 0:H] = xb[:, H:D]
  st1_ref[:, H:D] = xl[:, H:D]
  dt0_ref[...] = xr[:, 0:H]
  dt1_ref[...] = xr[:, H:D]


def _make_tables(x, W_l, b_l, W_r, b_r):
  R = 1000
  return pl.pallas_call(
      _tables_body,
      grid=(N // R,),
      in_specs=[
          pl.BlockSpec((R, D), lambda i: (i, 0)),
          pl.BlockSpec((D, D), lambda i: (0, 0)),
          pl.BlockSpec((D,), lambda i: (0,)),
          pl.BlockSpec((D, D), lambda i: (0, 0)),
          pl.BlockSpec((D,), lambda i: (0,)),
      ],
      out_specs=[
          pl.BlockSpec((R, D), lambda i: (i, 0)),
          pl.BlockSpec((R, D), lambda i: (i, 0)),
          pl.BlockSpec((R, H), lambda i: (i, 0)),
          pl.BlockSpec((R, H), lambda i: (i, 0)),
      ],
      out_shape=[
          jax.ShapeDtypeStruct((N, D), jnp.float32),
          jax.ShapeDtypeStruct((N, D), jnp.float32),
          jax.ShapeDtypeStruct((N, H), jnp.float32),
          jax.ShapeDtypeStruct((N, H), jnp.float32),
      ],
  )(x, W_l, b_l, W_r, b_r)


def _sc_body(st0, st1, dt0, dt1, srci, dsti, alpha, h0, h1,
             acc, sv0, sv1, sv2, sv3, dv0, dv1, dv2, dv3,
             g1a, g1b, g2a, g2b, aba, abb, oba, obb, hbuf,
             ix0, ix1, in0, in1, sc0, sc1):
  c = lax.axis_index("c")
  s = lax.axis_index("s")
  row0 = s * RPT
  eb = s * EPT
  sv = (sv0, sv1, sv2, sv3)
  dv = (dv0, dv1, dv2, dv3)
  g1 = (g1a, g1b)
  g2 = (g2a, g2b)
  ab = (aba, abb)
  ob = (oba, obb)
  ixs = (ix0, ix1)
  ins = (in0, in1)
  scs = (sc0, sc1)

  def gblk(i):
    return jnp.minimum(s * NBLK + i, NGB - 1)

  @pl.loop(0, RB * (D // L))
  def _zfill(k):
    r = k // (D // L)
    cc = (k % (D // L)) * L
    oba[r, pl.ds(cc, L)] = jnp.zeros((L,), jnp.float32)

  @pl.loop(0, RPT // RB)
  def _zcopy(j):
    pltpu.sync_copy(oba.at[pl.ds(0, RB)], acc.at[pl.ds(row0 + j * RB, RB)])

  pltpu.sync_copy(oba.at[pl.ds(0, RTAIL)],
                  acc.at[pl.ds(row0 + NRB * RB, RTAIL)])

  plsc.subcore_barrier()

  for half in range(NC):
    @pl.when(c == half)
    def _half_program():
      st = st0 if half == 0 else st1
      dt = dt0 if half == 0 else dt1
      coff = half * H

      def compute_block(gin, din, ain, oout):
        @plsc.parallel_loop(0, B, unroll=2)
        def _row(r):
          for q in range(H // (2 * L)):
            c32 = q * 2 * L
            ilv = plsc.PackFormat.INTERLEAVED
            xa, xb = plsc.unpack(gin[r, pl.ds(c32, 2 * L)], format=ilv)
            xla, xlb = plsc.unpack(gin[r, pl.ds(H + c32, 2 * L)], format=ilv)
            xra, xrb = plsc.unpack(din[r, pl.ds(c32, 2 * L)], format=ilv)
            for t, (xv, xlv, xrv) in enumerate(((xa, xla, xra),
                                                (xb, xlb, xrb))):
              cc = c32 + t * L
              av = ain[r, pl.ds(cc, L)]
              z = (xlv + xrv) * av
              z = jnp.maximum(z, SLOPE * z)
              p = jnp.exp(z)
              oout[r, pl.ds(cc, L)] = xv * p
              oout[r, pl.ds(H + cc, L)] = p

      def emit_block(i, u):
        s2 = u % 2
        n2 = (u + 1) % 2
        pltpu.make_async_copy(srci.at[pl.ds(0, B)], sv[(u + 1) % 4],
                              ixs[n2]).wait()
        pltpu.make_async_copy(dsti.at[pl.ds(0, B)], dv[(u + 1) % 4],
                              ixs[n2]).wait()
        bn = gblk(i + 1) * B
        pltpu.async_copy(st.at[sv[(u + 1) % 4]], g1[n2], ins[n2])
        pltpu.async_copy(dt.at[dv[(u + 1) % 4]], g2[n2], ins[n2])
        pltpu.async_copy(alpha.at[pl.ds(bn, B), pl.ds(coff, H)],
                         ab[n2], ins[n2])
        pltpu.make_async_copy(st.at[pl.ds(0, B)], g1[s2], ins[s2]).wait()
        pltpu.make_async_copy(dt.at[pl.ds(0, B)], g2[s2], ins[s2]).wait()
        pltpu.make_async_copy(alpha.at[pl.ds(0, B), pl.ds(0, H)],
                              ab[s2], ins[s2]).wait()
        pltpu.make_async_copy(alpha.at[pl.ds(0, B), :], ob[s2],
                              scs[s2]).wait()
        compute_block(g1[s2], g2[s2], ab[s2], ob[s2])
        pltpu.async_copy(ob[s2], acc.at[dv[u % 4]], scs[s2], add=True)
        bn2 = gblk(i + 2) * B
        pltpu.async_copy(srci.at[pl.ds(bn2, B)], sv[(u + 2) % 4], ixs[s2])
        pltpu.async_copy(dsti.at[pl.ds(bn2, B)], dv[(u + 2) % 4], ixs[s2])

      pltpu.sync_copy(srci.at[pl.ds(eb, B)], sv0)
      pltpu.sync_copy(dsti.at[pl.ds(eb, B)], dv0)
      pltpu.async_copy(st.at[sv0], g1a, in0)
      pltpu.async_copy(dt.at[dv0], g2a, in0)
      pltpu.async_copy(alpha.at[pl.ds(eb, B), pl.ds(coff, H)], aba, in0)
      b1 = gblk(1) * B
      pltpu.async_copy(srci.at[pl.ds(b1, B)], sv1, ix1)
      pltpu.async_copy(dsti.at[pl.ds(b1, B)], dv1, ix1)
      pltpu.async_copy(alpha.at[pl.ds(0, B), :], oba, sc0)
      pltpu.async_copy(alpha.at[pl.ds(0, B), :], obb, sc1)

      @pl.loop(0, NJ)
      def _super(j):
        i0 = j * U
        for u in range(U):
          emit_block(i0 + u, u)

      emit_block(NB_MAIN, 0)
      emit_block(NB_MAIN + 1, 1)

      pltpu.make_async_copy(alpha.at[pl.ds(0, B), :], ob[0], sc0).wait()
      pltpu.make_async_copy(alpha.at[pl.ds(0, B), :], ob[1], sc1).wait()
      pltpu.make_async_copy(st.at[pl.ds(0, B)], g1[0], in0).wait()
      pltpu.make_async_copy(dt.at[pl.ds(0, B)], g2[0], in0).wait()
      pltpu.make_async_copy(alpha.at[pl.ds(0, B), pl.ds(0, H)],
                            ab[0], in0).wait()
      pltpu.make_async_copy(srci.at[pl.ds(0, B)], sv3, ix1).wait()
      pltpu.make_async_copy(dsti.at[pl.ds(0, B)], dv3, ix1).wait()

      plsc.subcore_barrier()

      hout = h0 if half == 0 else h1

      def div_chunk(rows, roff):
        pltpu.sync_copy(acc.at[pl.ds(roff, rows)], oba.at[pl.ds(0, rows)])

        @pl.loop(0, rows)
        def _erow(r):
          for q in range(H // L):
            cc = q * L
            num = oba[r, pl.ds(cc, L)]
            den = oba[r, pl.ds(H + cc, L)]
            hbuf[r, pl.ds(cc, L)] = jnp.where(den > 0.0, num / den, 0.0)

        pltpu.sync_copy(hbuf.at[pl.ds(0, rows)], hout.at[pl.ds(roff, rows)])

      @pl.loop(0, NRB)
      def _ep(k):
        div_chunk(RB, row0 + k * RB)

      div_chunk(RTAIL, row0 + NRB * RB)


_sc_edge = functools.partial(
    pl.kernel,
    out_type=[jax.ShapeDtypeStruct((N, H), jnp.float32),
              jax.ShapeDtypeStruct((N, H), jnp.float32)],
    mesh=plsc.VectorSubcoreMesh(core_axis_name="c", subcore_axis_name="s",
                                num_cores=NC, num_subcores=NS),
    compiler_params=pltpu.CompilerParams(use_tc_tiling_on_sc=False,
                                         needs_layout_passes=False),
    scratch_types=[
        pltpu.VMEM_SHARED((N, D), jnp.float32),
        pltpu.VMEM((B,), jnp.int32),
        pltpu.VMEM((B,), jnp.int32),
        pltpu.VMEM((B,), jnp.int32),
        pltpu.VMEM((B,), jnp.int32),
        pltpu.VMEM((B,), jnp.int32),
        pltpu.VMEM((B,), jnp.int32),
        pltpu.VMEM((B,), jnp.int32),
        pltpu.VMEM((B,), jnp.int32),
        pltpu.VMEM((B, D), jnp.bfloat16),
        pltpu.VMEM((B, D), jnp.bfloat16),
        pltpu.VMEM((B, H), jnp.bfloat16),
        pltpu.VMEM((B, H), jnp.bfloat16),
        pltpu.VMEM((B, H), jnp.float32),
        pltpu.VMEM((B, H), jnp.float32),
        pltpu.VMEM((B, D), jnp.float32),
        pltpu.VMEM((B, D), jnp.float32),
        pltpu.VMEM((RB, H), jnp.float32),
        pltpu.SemaphoreType.DMA,
        pltpu.SemaphoreType.DMA,
        pltpu.SemaphoreType.DMA,
        pltpu.SemaphoreType.DMA,
        pltpu.SemaphoreType.DMA,
        pltpu.SemaphoreType.DMA,
    ],
)(_sc_body)


def _ileave(t):
  n, w = t.shape
  t = t.reshape(n, w // 32, 2, 16).transpose(0, 1, 3, 2).reshape(n, w)
  return t.astype(jnp.bfloat16)


def kernel(x, edge_index, alpha, ntype, etype, W_l, b_l, W_r, b_r):
  ei = edge_index.astype(jnp.int32)
  st0, st1, dt0, dt1 = _make_tables(x, W_l, b_l, W_r, b_r)
  h0, h1 = _sc_edge(_ileave(st0), _ileave(st1), _ileave(dt0), _ileave(dt1),
                    ei[0], ei[1], alpha)
  return jnp.concatenate([h0, h1], axis=1)
